# baseline (speedup 1.0000x reference)
import functools

import jax
import jax.numpy as jnp
from jax import lax
from jax.experimental import pallas as pl
from jax.experimental.pallas import tpu as pltpu
from jax.experimental.pallas import tpu_sc as plsc

B_ROWS = 16384
NJ = 20
DIM = 64
NC = 2
NS = 16
NW = NC * NS

BLK = 256
T_PER_J = B_ROWS // BLK
BLOCKS = NJ * T_PER_J
BLK_PER_W = BLOCKS // NW

_mesh = plsc.VectorSubcoreMesh(core_axis_name="c", subcore_axis_name="s")


@functools.partial(
    pl.kernel,
    out_type=jax.ShapeDtypeStruct((NJ, DIM // 8, B_ROWS // 128, 8, 128), jnp.float32),
    mesh=_mesh,
    scratch_types=[
        pltpu.VMEM((4, 2, 128), jnp.int32),
        pltpu.VMEM((4, BLK, DIM), jnp.float32),
        pltpu.VMEM((2, 8, BLK // 128, 12, 129), jnp.float32),
        pltpu.SemaphoreType.DMA,
        pltpu.SemaphoreType.DMA,
        pltpu.SemaphoreType.DMA,
        pltpu.SemaphoreType.DMA,
        pltpu.SemaphoreType.DMA,
        pltpu.SemaphoreType.DMA,
    ],
    compiler_params=pltpu.CompilerParams(
        use_tc_tiling_on_sc=False, needs_layout_passes=False
    ),
)
def _embed(
    x_hbm, table_hbm, out_hbm, idx_v, rows_v, tp_v,
    gsem0, gsem1, gsem2, gsem3, wsem0, wsem1,
):
    wid = lax.axis_index("s") * NC + lax.axis_index("c")
    gsems = (gsem0, gsem1, gsem2, gsem3)
    wsems = (wsem0, wsem1)
    iota16 = lax.iota(jnp.int32, 16)

    def fire(k, db):
        b = wid * BLK_PER_W + k
        j = b // T_PER_J
        t = b % T_PER_J
        pltpu.sync_copy(
            x_hbm.at[j // 8, pl.ds(2 * t, 2), j % 8], idx_v.at[db]
        )
        for c in range(2):
            pltpu.async_copy(
                table_hbm.at[idx_v.at[db, c]],
                rows_v.at[db, pl.ds(128 * c, 128)],
                gsems[db],
            )

    s_vec = iota16 & 7
    ts_vecs = [(iota16 >> 3) + 2 * k for k in range(4)]

    def transpose_block(db, tb):
        tbv = jnp.full((16,), tb, jnp.int32)

        @plsc.parallel_loop(0, 128, unroll=4)
        def _(i0):
            lv = jnp.full((16,), i0, jnp.int32)
            for tlq in range(BLK // 128):
                row = i0 + 128 * tlq
                tv = jnp.full((16,), tlq, jnp.int32)
                for k in range(4):
                    vals = rows_v[db, row, pl.ds(16 * k, 16)]
                    plsc.store_scatter(
                        tp_v, [tbv, ts_vecs[k], tv, s_vec, lv], vals
                    )

    def run_block(k, db):
        for _ in range(2):
            pltpu.make_async_copy(
                table_hbm.at[idx_v.at[db, 0]],
                rows_v.at[db, pl.ds(0, 128)],
                gsems[db],
            ).wait()

        @pl.when(k + 2 < BLK_PER_W)
        def _():
            fire(k + 2, (db + 2) % 4)

        tb = db % 2
        @pl.when(k >= 2)
        def _():
            pltpu.make_async_copy(
                tp_v.at[tb, :, :, pl.ds(0, 8), pl.ds(0, 128)], out_hbm.at[0, :, pl.ds(0, 2)], wsems[tb]
            ).wait()

        transpose_block(db, tb)

        b = wid * BLK_PER_W + k
        j = b // T_PER_J
        t = b % T_PER_J
        pltpu.async_copy(
            tp_v.at[tb, :, :, pl.ds(0, 8), pl.ds(0, 128)],
            out_hbm.at[j, :, pl.ds(2 * t, 2)],
            wsems[tb],
        )

    fire(0, 0)
    fire(1, 1)

    def quad(i, carry):
        for db in range(4):
            run_block(i * 4 + db, db)
        return carry

    lax.fori_loop(0, BLK_PER_W // 4, quad, 0)

    for tb in range(2):
        pltpu.make_async_copy(
            tp_v.at[tb, :, :, pl.ds(0, 8), pl.ds(0, 128)], out_hbm.at[0, :, pl.ds(0, 2)], wsems[tb]
        ).wait()


def kernel(x, table):
    xp = jnp.pad(x.astype(jnp.int32), ((0, 0), (0, 4)))
    xi = xp.T.reshape(3, 8, B_ROWS // 128, 128).transpose(0, 2, 1, 3)
    out5 = _embed(xi, table)
    return out5.transpose(2, 4, 0, 1, 3).reshape(B_ROWS, NJ, DIM)


# speedup vs baseline: 1.0222x; 1.0222x over previous
import functools

import jax
import jax.numpy as jnp
from jax import lax
from jax.experimental import pallas as pl
from jax.experimental.pallas import tpu as pltpu
from jax.experimental.pallas import tpu_sc as plsc

B_ROWS = 16384
NJ = 20
DIM = 64
NC = 2
NS = 16
NW = NC * NS

BLK = 256
T_PER_J = B_ROWS // BLK
BLOCKS = NJ * T_PER_J
BLK_PER_W = BLOCKS // NW

_mesh = plsc.VectorSubcoreMesh(core_axis_name="c", subcore_axis_name="s")


@functools.partial(
    pl.kernel,
    out_type=jax.ShapeDtypeStruct((NJ, DIM // 8, B_ROWS // 128, 8, 128), jnp.float32),
    mesh=_mesh,
    scratch_types=[
        pltpu.VMEM((2, 2, 128), jnp.int32),
        pltpu.VMEM((2, BLK, 128), jnp.float32),
        pltpu.VMEM((2, 8, BLK // 128, 12, 129), jnp.float32),
        pltpu.SemaphoreType.DMA,
        pltpu.SemaphoreType.DMA,
        pltpu.SemaphoreType.DMA,
    ],
    compiler_params=pltpu.CompilerParams(
        use_tc_tiling_on_sc=False, needs_layout_passes=False
    ),
)
def _embed(
    x_hbm, table_hbm, out_hbm, idx_v, rows_v, tp_v, gsem, wsem0, wsem1,
):
    wid = lax.axis_index("s") * NC + lax.axis_index("c")
    wsems = (wsem0, wsem1)
    iota16 = lax.iota(jnp.int32, 16)

    def fire(k, db):
        b = wid * BLK_PER_W + k
        j = b // T_PER_J
        t = b % T_PER_J
        pltpu.sync_copy(
            x_hbm.at[j // 8, pl.ds(2 * t, 2), j % 8], idx_v.at[db]
        )
        for c in range(2):
            pltpu.async_copy(
                table_hbm.at[idx_v.at[db, c]],
                rows_v.at[db, pl.ds(128 * c, 128)],
                gsem,
            )

    s_vec = iota16 & 7
    ts_vecs = [(iota16 >> 3) + 2 * k for k in range(4)]

    def transpose_block(db, tb):
        tbv = jnp.full((16,), tb, jnp.int32)

        @plsc.parallel_loop(0, 128, unroll=4)
        def _(i0):
            lv = jnp.full((16,), i0, jnp.int32)
            for tlq in range(BLK // 128):
                row = i0 + 128 * tlq
                tv = jnp.full((16,), tlq, jnp.int32)
                for k in range(4):
                    vals = rows_v[db, row, pl.ds(16 * k, 16)]
                    plsc.store_scatter(
                        tp_v, [tbv, ts_vecs[k], tv, s_vec, lv], vals
                    )

    def run_block(k, db):
        for _ in range(2):
            pltpu.make_async_copy(
                table_hbm.at[idx_v.at[db, 0]],
                rows_v.at[db, pl.ds(0, 128)],
                gsem,
            ).wait()

        @pl.when(k + 1 < BLK_PER_W)
        def _():
            fire(k + 1, 1 - db)

        tb = db
        @pl.when(k >= 2)
        def _():
            pltpu.make_async_copy(
                tp_v.at[tb, :, :, pl.ds(0, 8), pl.ds(0, 128)], out_hbm.at[0, :, pl.ds(0, 2)], wsems[tb]
            ).wait()

        transpose_block(db, tb)

        b = wid * BLK_PER_W + k
        j = b // T_PER_J
        t = b % T_PER_J
        pltpu.async_copy(
            tp_v.at[tb, :, :, pl.ds(0, 8), pl.ds(0, 128)],
            out_hbm.at[j, :, pl.ds(2 * t, 2)],
            wsems[tb],
        )

    fire(0, 0)

    def pair(i, carry):
        for db in range(2):
            run_block(i * 2 + db, db)
        return carry

    lax.fori_loop(0, BLK_PER_W // 2, pair, 0)

    for tb in range(2):
        pltpu.make_async_copy(
            tp_v.at[tb, :, :, pl.ds(0, 8), pl.ds(0, 128)], out_hbm.at[0, :, pl.ds(0, 2)], wsems[tb]
        ).wait()


def kernel(x, table):
    xp = jnp.pad(x.astype(jnp.int32), ((0, 0), (0, 4)))
    xi = xp.T.reshape(3, 8, B_ROWS // 128, 128).transpose(0, 2, 1, 3)
    tp = jnp.pad(table, ((0, 0), (0, 128 - DIM)))
    out5 = _embed(xi, tp)
    return out5.transpose(2, 4, 0, 1, 3).reshape(B_ROWS, NJ, DIM)


# speedup vs baseline: 1.0814x; 1.0580x over previous
import functools

import jax
import jax.numpy as jnp
from jax import lax
from jax.experimental import pallas as pl
from jax.experimental.pallas import tpu as pltpu
from jax.experimental.pallas import tpu_sc as plsc

B_ROWS = 16384
NJ = 20
DIM = 64
NC = 2
NS = 16
NW = NC * NS

BLK = 256
T_PER_J = B_ROWS // BLK
BLOCKS = NJ * T_PER_J
BLK_PER_W = BLOCKS // NW

_mesh = plsc.VectorSubcoreMesh(core_axis_name="c", subcore_axis_name="s")


@functools.partial(
    pl.kernel,
    out_type=jax.ShapeDtypeStruct((NJ, DIM // 8, B_ROWS // 128, 8, 128), jnp.float32),
    mesh=_mesh,
    scratch_types=[
        pltpu.VMEM((2, 2, 128), jnp.int32),
        pltpu.VMEM((2, BLK, DIM), jnp.float32),
        pltpu.VMEM((2, 8, BLK // 128, 12, 129), jnp.float32),
        pltpu.SemaphoreType.DMA,
        pltpu.SemaphoreType.DMA,
        pltpu.SemaphoreType.DMA,
    ],
    compiler_params=pltpu.CompilerParams(
        use_tc_tiling_on_sc=False, needs_layout_passes=False
    ),
)
def _embed(
    x_hbm, table_hbm, out_hbm, idx_v, rows_v, tp_v, gsem, wsem0, wsem1,
):
    wid = lax.axis_index("s") * NC + lax.axis_index("c")
    wsems = (wsem0, wsem1)
    iota16 = lax.iota(jnp.int32, 16)

    def fire(k, db):
        b = wid * BLK_PER_W + k
        j = b // T_PER_J
        t = b % T_PER_J
        pltpu.sync_copy(
            x_hbm.at[j // 8, pl.ds(2 * t, 2), j % 8], idx_v.at[db]
        )
        for c in range(2):
            pltpu.async_copy(
                table_hbm.at[idx_v.at[db, c]],
                rows_v.at[db, pl.ds(128 * c, 128)],
                gsem,
            )

    s_vec = iota16 & 7
    ts_vecs = [(iota16 >> 3) + 2 * k for k in range(4)]

    def transpose_block(db, tb):
        tbv = jnp.full((16,), tb, jnp.int32)

        @plsc.parallel_loop(0, 128, unroll=4)
        def _(i0):
            lv = jnp.full((16,), i0, jnp.int32)
            for tlq in range(BLK // 128):
                row = i0 + 128 * tlq
                tv = jnp.full((16,), tlq, jnp.int32)
                for k in range(4):
                    vals = rows_v[db, row, pl.ds(16 * k, 16)]
                    plsc.store_scatter(
                        tp_v, [tbv, ts_vecs[k], tv, s_vec, lv], vals
                    )

    def run_block(k, db):
        for _ in range(2):
            pltpu.make_async_copy(
                table_hbm.at[idx_v.at[db, 0]],
                rows_v.at[db, pl.ds(0, 128)],
                gsem,
            ).wait()

        @pl.when(k + 1 < BLK_PER_W)
        def _():
            fire(k + 1, 1 - db)

        tb = db
        @pl.when(k >= 2)
        def _():
            pltpu.make_async_copy(
                tp_v.at[tb, :, :, pl.ds(0, 8), pl.ds(0, 128)], out_hbm.at[0, :, pl.ds(0, 2)], wsems[tb]
            ).wait()

        transpose_block(db, tb)

        b = wid * BLK_PER_W + k
        j = b // T_PER_J
        t = b % T_PER_J
        pltpu.async_copy(
            tp_v.at[tb, :, :, pl.ds(0, 8), pl.ds(0, 128)],
            out_hbm.at[j, :, pl.ds(2 * t, 2)],
            wsems[tb],
        )

    fire(0, 0)

    def pair(i, carry):
        for db in range(2):
            run_block(i * 2 + db, db)
        return carry

    lax.fori_loop(0, BLK_PER_W // 2, pair, 0)

    for tb in range(2):
        pltpu.make_async_copy(
            tp_v.at[tb, :, :, pl.ds(0, 8), pl.ds(0, 128)], out_hbm.at[0, :, pl.ds(0, 2)], wsems[tb]
        ).wait()


def kernel(x, table):
    xp = jnp.pad(x.astype(jnp.int32) * 2, ((0, 0), (0, 4)))
    xi = xp.T.reshape(3, 8, B_ROWS // 128, 128).transpose(0, 2, 1, 3)
    tp = jnp.pad(table, ((0, 0), (0, 128 - DIM))).reshape(2 * 1000000, DIM)
    out5 = _embed(xi, tp)
    return out5.transpose(2, 4, 0, 1, 3).reshape(B_ROWS, NJ, DIM)


# speedup vs baseline: 1.1027x; 1.0197x over previous
import functools

import jax
import jax.numpy as jnp
from jax import lax
from jax.experimental import pallas as pl
from jax.experimental.pallas import tpu as pltpu
from jax.experimental.pallas import tpu_sc as plsc

B_ROWS = 16384
NJ = 20
DIM = 64
NC = 2
NS = 16
NW = NC * NS

BLK = 256
T_PER_J = B_ROWS // BLK
BLOCKS = NJ * T_PER_J
BLK_PER_W = BLOCKS // NW

_mesh = plsc.VectorSubcoreMesh(core_axis_name="c", subcore_axis_name="s")


@functools.partial(
    pl.kernel,
    out_type=jax.ShapeDtypeStruct((NJ, DIM // 8, B_ROWS // 128, 8, 128), jnp.float32),
    mesh=_mesh,
    scratch_types=[
        pltpu.VMEM((4, 2, 128), jnp.int32),
        pltpu.VMEM((4, BLK, DIM), jnp.float32),
        pltpu.VMEM((2, 8, BLK // 128, 12, 129), jnp.float32),
        pltpu.SemaphoreType.DMA,
        pltpu.SemaphoreType.DMA,
        pltpu.SemaphoreType.DMA,
        pltpu.SemaphoreType.DMA,
        pltpu.SemaphoreType.DMA,
        pltpu.SemaphoreType.DMA,
    ],
    compiler_params=pltpu.CompilerParams(
        use_tc_tiling_on_sc=False, needs_layout_passes=False
    ),
)
def _embed(
    x_hbm, table_hbm, out_hbm, idx_v, rows_v, tp_v,
    gsem0, gsem1, gsem2, gsem3, wsem0, wsem1,
):
    wid = lax.axis_index("s") * NC + lax.axis_index("c")
    gsems = (gsem0, gsem1, gsem2, gsem3)
    wsems = (wsem0, wsem1)
    iota16 = lax.iota(jnp.int32, 16)

    def fire(k, db):
        b = wid * BLK_PER_W + k
        j = b // T_PER_J
        t = b % T_PER_J
        pltpu.sync_copy(
            x_hbm.at[j // 8, pl.ds(2 * t, 2), j % 8], idx_v.at[db]
        )
        for c in range(2):
            pltpu.async_copy(
                table_hbm.at[idx_v.at[db, c]],
                rows_v.at[db, pl.ds(128 * c, 128)],
                gsems[db],
            )

    s_vec = iota16 & 7
    ts_vecs = [(iota16 >> 3) + 2 * k for k in range(4)]

    def transpose_block(db, tb):
        tbv = jnp.full((16,), tb, jnp.int32)

        @plsc.parallel_loop(0, 128, unroll=4)
        def _(i0):
            lv = jnp.full((16,), i0, jnp.int32)
            for tlq in range(BLK // 128):
                row = i0 + 128 * tlq
                tv = jnp.full((16,), tlq, jnp.int32)
                for k in range(4):
                    vals = rows_v[db, row, pl.ds(16 * k, 16)]
                    plsc.store_scatter(
                        tp_v, [tbv, ts_vecs[k], tv, s_vec, lv], vals
                    )

    def run_block(k, db):
        for _ in range(2):
            pltpu.make_async_copy(
                table_hbm.at[idx_v.at[db, 0]],
                rows_v.at[db, pl.ds(0, 128)],
                gsems[db],
            ).wait()

        @pl.when(k + 2 < BLK_PER_W)
        def _():
            fire(k + 2, (db + 2) % 4)

        tb = db % 2
        @pl.when(k >= 2)
        def _():
            pltpu.make_async_copy(
                tp_v.at[tb, :, :, pl.ds(0, 8), pl.ds(0, 128)], out_hbm.at[0, :, pl.ds(0, 2)], wsems[tb]
            ).wait()

        transpose_block(db, tb)

        b = wid * BLK_PER_W + k
        j = b // T_PER_J
        t = b % T_PER_J
        pltpu.async_copy(
            tp_v.at[tb, :, :, pl.ds(0, 8), pl.ds(0, 128)],
            out_hbm.at[j, :, pl.ds(2 * t, 2)],
            wsems[tb],
        )

    fire(0, 0)
    fire(1, 1)

    def quad(i, carry):
        for db in range(4):
            run_block(i * 4 + db, db)
        return carry

    lax.fori_loop(0, BLK_PER_W // 4, quad, 0)

    for tb in range(2):
        pltpu.make_async_copy(
            tp_v.at[tb, :, :, pl.ds(0, 8), pl.ds(0, 128)], out_hbm.at[0, :, pl.ds(0, 2)], wsems[tb]
        ).wait()


def kernel(x, table):
    xp = jnp.pad(x.astype(jnp.int32) * 2, ((0, 0), (0, 4)))
    xi = xp.T.reshape(3, 8, B_ROWS // 128, 128).transpose(0, 2, 1, 3)
    tp = jnp.pad(table, ((0, 0), (0, 128 - DIM))).reshape(2 * 1000000, DIM)
    out5 = _embed(xi, tp)
    return out5.transpose(2, 4, 0, 1, 3).reshape(B_ROWS, NJ, DIM)
